# Initial kernel scaffold; baseline (speedup 1.0000x reference)
import jax
import jax.numpy as jnp
from jax.experimental import pallas as pl


def kernel(x, edge_index, edge_weight, ent_emb, rel_trans):
    raise NotImplementedError("write your pallas kernel here")


# speedup vs baseline: 3.8632x; 3.8632x over previous
import functools

import jax
import jax.numpy as jnp
from jax import lax
from jax.experimental import pallas as pl
from jax.experimental.pallas import tpu as pltpu
from jax.experimental.pallas import tpu_sc as plsc

CH = 80
NSUB = 16
NCORE = 2


def _dense_combine(acc, w, relu, normalize):
  g_dim, n, d = acc.shape
  blk = 1000
  assert n % blk == 0

  def body(acc_ref, w_ref, o_ref):
    s = jnp.zeros((blk, d), jnp.float32)
    for g in range(g_dim):
      s = s + jnp.dot(acc_ref[g], w_ref[g], preferred_element_type=jnp.float32)
    if relu:
      s = jnp.maximum(s, 0.0)
    if normalize:
      nrm = jnp.sqrt(jnp.sum(s * s, axis=1, keepdims=True))
      s = s / jnp.maximum(nrm, 1e-12)
    o_ref[...] = s

  return pl.pallas_call(
      body,
      grid=(n // blk,),
      in_specs=[
          pl.BlockSpec((g_dim, blk, d), lambda i: (0, i, 0)),
          pl.BlockSpec((g_dim, d, d), lambda i: (0, 0, 0)),
      ],
      out_specs=pl.BlockSpec((blk, d), lambda i: (i, 0)),
      out_shape=jax.ShapeDtypeStruct((n, d), jnp.float32),
  )(acc, w)


def _sc_scatter(emb, src_flat, dst_flat, w_flat, zeros_nd, n_rel, n_edge):
  n, d = emb.shape
  n_chunks = n_edge // CH
  assert n_edge % CH == 0
  row_chunk = 40
  n_row_chunks = n // row_chunk
  assert n % row_chunk == 0
  rel_per_core = n_rel // NCORE
  mesh = plsc.VectorSubcoreMesh(core_axis_name="c", subcore_axis_name="s")

  @functools.partial(
      pl.kernel,
      mesh=mesh,
      out_type=jax.ShapeDtypeStruct((n_rel * n, d), jnp.float32),
      scratch_types=[
          pltpu.VMEM((CH,), jnp.int32),
          pltpu.VMEM((CH,), jnp.int32),
          pltpu.VMEM((CH,), jnp.float32),
          pltpu.VMEM((CH, d), jnp.float32),
          pltpu.VMEM_SHARED((n, d), jnp.float32),
          pltpu.SemaphoreType.DMA,
      ],
  )
  def k(emb_hbm, src_hbm, dst_hbm, w_hbm, zero_hbm, out_hbm,
        src_v, dst_v, w_v, rows_v, acc_sh, sem):
    c = lax.axis_index("c")
    s = lax.axis_index("s")
    n_row_mine = (n_row_chunks - s + NSUB - 1) // NSUB

    for r_local in range(rel_per_core):
      r = c * rel_per_core + r_local

      def zero_body(i, _):
        row0 = (s + i * NSUB) * row_chunk
        pltpu.sync_copy(zero_hbm.at[pl.ds(row0, row_chunk)],
                        acc_sh.at[pl.ds(row0, row_chunk)])
        return 0

      lax.fori_loop(0, n_row_mine, zero_body, 0)
      plsc.subcore_barrier()

      n_mine = (n_chunks - s + NSUB - 1) // NSUB

      def chunk_body(i, _, r=r):
        cid = s + i * NSUB
        base = r * n_edge + cid * CH
        pltpu.sync_copy(src_hbm.at[pl.ds(base, CH)], src_v)
        pltpu.sync_copy(dst_hbm.at[pl.ds(base, CH)], dst_v)
        pltpu.sync_copy(w_hbm.at[pl.ds(base, CH)], w_v)
        pltpu.async_copy(emb_hbm.at[src_v], rows_v, sem).wait()

        def edge_body(b, _):
          wvec = w_v[pl.ds(b * 16, 16)]
          for t in range(16):
            e = b * 16 + t
            for j in range(d // 16):
              rows_v[e, pl.ds(j * 16, 16)] = (
                  rows_v[e, pl.ds(j * 16, 16)] * wvec[t])
          return 0

        lax.fori_loop(0, CH // 16, edge_body, 0)
        pltpu.sync_copy(rows_v, acc_sh.at[dst_v], add=True)
        return 0

      lax.fori_loop(0, n_mine, chunk_body, 0)
      plsc.subcore_barrier()

      def out_body(i, _, r=r):
        row0 = (s + i * NSUB) * row_chunk
        pltpu.sync_copy(acc_sh.at[pl.ds(row0, row_chunk)],
                        out_hbm.at[pl.ds(r * n + row0, row_chunk)])
        return 0

      lax.fori_loop(0, n_row_mine, out_body, 0)

  out = k(emb, src_flat, dst_flat, w_flat, zeros_nd)
  return out.reshape(n_rel, n, d)


def kernel(x, edge_index, edge_weight, ent_emb, rel_trans):
  n, _ = x.shape
  n_rel, _, n_edge = edge_index.shape
  n_layers = rel_trans.shape[0]
  d = ent_emb.shape[1]

  dst_flat = edge_index[:, 0, :].reshape(-1)
  src_flat = edge_index[:, 1, :].reshape(-1)
  w_flat = edge_weight.reshape(-1)
  zeros_nd = jnp.zeros((n, d), jnp.float32)
  w_t = jnp.transpose(rel_trans, (0, 1, 3, 2))

  emb = _dense_combine(x[None], ent_emb[None], relu=False, normalize=False)
  for l in range(n_layers):
    acc = _sc_scatter(emb, src_flat, dst_flat, w_flat, zeros_nd, n_rel, n_edge)
    emb = _dense_combine(acc, w_t[l], relu=True,
                         normalize=(l == n_layers - 1))
  return emb


# speedup vs baseline: 4.2007x; 1.0873x over previous
import functools

import jax
import jax.numpy as jnp
from jax import lax
from jax.experimental import pallas as pl
from jax.experimental.pallas import tpu as pltpu
from jax.experimental.pallas import tpu_sc as plsc

CH = 80
NSUB = 16
NCORE = 2


def _dense_combine(acc, w, relu, normalize):
  g_dim, n, d = acc.shape
  blk = 1000
  assert n % blk == 0

  def body(acc_ref, w_ref, o_ref):
    s = jnp.zeros((blk, d), jnp.float32)
    for g in range(g_dim):
      s = s + jnp.dot(acc_ref[g], w_ref[g], preferred_element_type=jnp.float32)
    if relu:
      s = jnp.maximum(s, 0.0)
    if normalize:
      nrm = jnp.sqrt(jnp.sum(s * s, axis=1, keepdims=True))
      s = s / jnp.maximum(nrm, 1e-12)
    o_ref[...] = s

  return pl.pallas_call(
      body,
      grid=(n // blk,),
      in_specs=[
          pl.BlockSpec((g_dim, blk, d), lambda i: (0, i, 0)),
          pl.BlockSpec((g_dim, d, d), lambda i: (0, 0, 0)),
      ],
      out_specs=pl.BlockSpec((blk, d), lambda i: (i, 0)),
      out_shape=jax.ShapeDtypeStruct((n, d), jnp.float32),
  )(acc, w)


SB = 8
NB = 3
LA = 2


def _sc_scatter(emb, src2d, dst2d, w2d, zeros128, n_rel, npc):
  n, d = emb.shape
  rel_per_core = n_rel // NCORE
  n_sup = npc // SB
  assert npc % SB == 0
  n_pad = ((n + 128 * NSUB - 1) // (128 * NSUB)) * (128 * NSUB)
  rps = n_pad // NSUB
  tail = n - (NSUB - 1) * rps
  assert 0 < tail <= rps and tail % 8 == 0
  mesh = plsc.VectorSubcoreMesh(core_axis_name="c", subcore_axis_name="s")

  @functools.partial(
      pl.kernel,
      mesh=mesh,
      out_type=jax.ShapeDtypeStruct((n_rel * n, d), jnp.float32),
      scratch_types=[
          pltpu.VMEM((2, SB, CH), jnp.int32),
          pltpu.VMEM((2, SB, CH), jnp.int32),
          pltpu.VMEM((2, SB, CH), jnp.float32),
          pltpu.VMEM((NB, CH, d), jnp.float32),
          pltpu.VMEM((64, d), jnp.float32),
          pltpu.VMEM_SHARED((n_pad, d), jnp.float32),
          pltpu.SemaphoreType.DMA((NB,)),
          pltpu.SemaphoreType.DMA((NB,)),
          pltpu.SemaphoreType.DMA((2,)),
      ],
  )
  def k(emb_hbm, src_hbm, dst_hbm, w_hbm, zero_hbm, out_hbm,
        src_v, dst_v, w_v, rows_v, zbuf, acc_sh, gsem, ssem, isem):
    c = lax.axis_index("c")
    s = lax.axis_index("s")
    pltpu.sync_copy(zero_hbm, zbuf)

    for r_local in range(rel_per_core):
      r = c * rel_per_core + r_local
      crow0 = r * (NSUB * npc) + s * npc

      for z in range(rps // 64):
        pltpu.sync_copy(zbuf, acc_sh.at[pl.ds(s * rps + z * 64, 64)])
      plsc.subcore_barrier()

      def idx_start(sup, slot):
        pltpu.async_copy(src_hbm.at[pl.ds(crow0 + sup * SB, SB)],
                         src_v.at[slot], isem.at[slot])
        pltpu.async_copy(dst_hbm.at[pl.ds(crow0 + sup * SB, SB)],
                         dst_v.at[slot], isem.at[slot])
        pltpu.async_copy(w_hbm.at[pl.ds(crow0 + sup * SB, SB)],
                         w_v.at[slot], isem.at[slot])

      def idx_wait(slot):
        pltpu.make_async_copy(src_hbm.at[pl.ds(crow0, SB)],
                              src_v.at[slot], isem.at[slot]).wait()
        pltpu.make_async_copy(dst_hbm.at[pl.ds(crow0, SB)],
                              dst_v.at[slot], isem.at[slot]).wait()
        pltpu.make_async_copy(w_hbm.at[pl.ds(crow0, SB)],
                              w_v.at[slot], isem.at[slot]).wait()

      def g_start(ci):
        pltpu.async_copy(emb_hbm.at[src_v.at[(ci // SB) % 2, ci % SB]],
                         rows_v.at[ci % NB], gsem.at[ci % NB])

      def g_wait(b):
        pltpu.make_async_copy(emb_hbm.at[src_v.at[0, 0]],
                              rows_v.at[b], gsem.at[b]).wait()

      def s_start(ci):
        pltpu.async_copy(rows_v.at[ci % NB],
                         acc_sh.at[dst_v.at[(ci // SB) % 2, ci % SB]],
                         ssem.at[ci % NB], add=True)

      def s_drain(b):
        pltpu.make_async_copy(rows_v.at[b], acc_sh.at[dst_v.at[0, 0]],
                              ssem.at[b]).wait()

      idx_start(0, 0)
      idx_wait(0)
      for ci0 in range(LA):
        g_start(jnp.int32(ci0))

      def chunk_iter(ci, _):
        sup = ci // SB
        kk = ci % SB
        slot = sup % 2
        b = ci % NB

        @pl.when(jnp.logical_and(kk == 0, sup < n_sup - 1))
        def _():
          idx_start(sup + 1, (sup + 1) % 2)

        @pl.when(jnp.logical_and(kk == SB // 2, sup < n_sup - 1))
        def _():
          idx_wait((sup + 1) % 2)

        g_wait(b)

        la = ci + LA

        @pl.when(la < npc)
        def _():
          @pl.when(ci >= 1)
          def _():
            s_drain(la % NB)

          g_start(la)

        for t16 in range(CH // 16):
          wvec = w_v[slot, kk, pl.ds(t16 * 16, 16)]
          for t in range(16):
            e = t16 * 16 + t
            for j in range(d // 16):
              rows_v[b, e, pl.ds(j * 16, 16)] = (
                  rows_v[b, e, pl.ds(j * 16, 16)] * wvec[t])

        s_start(ci)
        return 0

      lax.fori_loop(0, npc, chunk_iter, 0)
      for b in range(NB):
        s_drain(b)
      plsc.subcore_barrier()

      @pl.when(s < NSUB - 1)
      def _():
        pltpu.sync_copy(acc_sh.at[pl.ds(s * rps, rps)],
                        out_hbm.at[pl.ds(r * n + s * rps, rps)])

      @pl.when(s == NSUB - 1)
      def _():
        pltpu.sync_copy(acc_sh.at[pl.ds((NSUB - 1) * rps, tail)],
                        out_hbm.at[pl.ds(r * n + (NSUB - 1) * rps, tail)])

  out = k(emb, src2d, dst2d, w2d, zeros128)
  return out.reshape(n_rel, n, d)


def kernel(x, edge_index, edge_weight, ent_emb, rel_trans):
  n, _ = x.shape
  n_rel, _, n_edge = edge_index.shape
  n_layers = rel_trans.shape[0]
  d = ent_emb.shape[1]

  per_sub = ((n_edge + NSUB * SB * CH - 1) // (NSUB * SB * CH)) * SB * CH
  e_pad = per_sub * NSUB
  npc = per_sub // CH
  pad = e_pad - n_edge
  src2d = jnp.pad(edge_index[:, 1, :], ((0, 0), (0, pad))).reshape(-1, CH)
  dst2d = jnp.pad(edge_index[:, 0, :], ((0, 0), (0, pad))).reshape(-1, CH)
  w2d = jnp.pad(edge_weight, ((0, 0), (0, pad))).reshape(-1, CH)
  zeros128 = jnp.zeros((64, d), jnp.float32)
  w_t = jnp.transpose(rel_trans, (0, 1, 3, 2))

  emb = _dense_combine(x[None], ent_emb[None], relu=False, normalize=False)
  for l in range(n_layers):
    acc = _sc_scatter(emb, src2d, dst2d, w2d, zeros128, n_rel, npc)
    emb = _dense_combine(acc, w_t[l], relu=True,
                         normalize=(l == n_layers - 1))
  return emb
